# baseline (speedup 1.0000x reference)
import functools

import jax
import jax.numpy as jnp
from jax import lax
from jax.experimental import pallas as pl
from jax.experimental.pallas import tpu as pltpu
from jax.experimental.pallas import tpu_sc as plsc

B = 4096
L = 200
HID = 64

VOCAB_PAIRS = 500000

NC = 2
NS = 16
NW = NC * NS
BB = B // NW
NBUF = 4
NT = L // NBUF


@functools.partial(
    pl.kernel,
    mesh=plsc.VectorSubcoreMesh(core_axis_name="c", subcore_axis_name="s"),
    compiler_params=pltpu.CompilerParams(
        use_tc_tiling_on_sc=False, needs_layout_passes=False),
    out_type=jax.ShapeDtypeStruct((L, HID // 8, NW, 8 * 128), jnp.float32),
    scratch_types=[
        pltpu.VMEM((L, HID), jnp.float32),
        pltpu.VMEM((BB,), jnp.int32),
        pltpu.VMEM((BB,), jnp.int32),
        pltpu.VMEM((BB,), jnp.int32),
        pltpu.VMEM((BB,), jnp.int32),
        pltpu.VMEM((BB,), jnp.int32),
        pltpu.VMEM((BB,), jnp.int32),
        pltpu.VMEM((BB,), jnp.int32),
        pltpu.VMEM((BB,), jnp.int32),
        pltpu.VMEM((BB, 2 * HID), jnp.float32),
        pltpu.VMEM((BB, 2 * HID), jnp.float32),
        pltpu.VMEM((BB, 2 * HID), jnp.float32),
        pltpu.VMEM((BB, 2 * HID), jnp.float32),
        pltpu.VMEM((HID // 8, 8 * 128), jnp.float32),
        pltpu.VMEM((HID // 8, 8 * 128), jnp.float32),
        pltpu.VMEM((HID // 8, 8 * 128), jnp.float32),
        pltpu.VMEM((HID // 8, 8 * 128), jnp.float32),
        pltpu.SemaphoreType.DMA,
        pltpu.SemaphoreType.DMA,
        pltpu.SemaphoreType.DMA,
        pltpu.SemaphoreType.DMA,
        pltpu.SemaphoreType.DMA,
        pltpu.SemaphoreType.DMA,
        pltpu.SemaphoreType.DMA,
        pltpu.SemaphoreType.DMA,
        pltpu.SemaphoreType.DMA,
        pltpu.SemaphoreType.DMA,
        pltpu.SemaphoreType.DMA,
        pltpu.SemaphoreType.DMA,
    ],
)
def _emb_kernel(xT_hbm, tok_hbm, pos_hbm, out_hbm,
                pos_v,
                ib0, ib1, ib2, ib3,
                ip0, ip1, ip2, ip3,
                gb0, gb1, gb2, gb3,
                sb0, sb1, sb2, sb3,
                si0, si1, si2, si3,
                sg0, sg1, sg2, sg3,
                ss0, ss1, ss2, ss3):
    ibs = (ib0, ib1, ib2, ib3)
    ips = (ip0, ip1, ip2, ip3)
    gbs = (gb0, gb1, gb2, gb3)
    sbs = (sb0, sb1, sb2, sb3)
    sis = (si0, si1, si2, si3)
    sgs = (sg0, sg1, sg2, sg3)
    sss = (ss0, ss1, ss2, ss3)

    wid = lax.axis_index("s") * NC + lax.axis_index("c")
    col0 = wid * BB

    pltpu.sync_copy(pos_hbm.at[pl.ds(0, L)], pos_v)

    iota = lax.iota(jnp.int32, 16)
    rows = [iota + 16 * j for j in range(8)]

    def issue_idx(g, k):
        pltpu.async_copy(xT_hbm.at[g, pl.ds(col0, BB)], ibs[k], sis[k])

    def wait_idx(k):
        pltpu.make_async_copy(
            xT_hbm.at[0, pl.ds(0, BB)], ibs[k], sis[k]).wait()

    def issue_gather(k):
        for r in range(BB // 16):
            ips[k][pl.ds(16 * r, 16)] = ibs[k][pl.ds(16 * r, 16)] >> 1
        pltpu.async_copy(tok_hbm.at[ips[k]], gbs[k], sgs[k])

    def wait_gather(k):
        pltpu.make_async_copy(
            tok_hbm.at[pl.ds(0, BB)], gbs[k], sgs[k]).wait()

    def issue_store(g, k):
        pltpu.async_copy(sbs[k], out_hbm.at[g, :, wid], sss[k])

    def wait_store(k):
        pltpu.make_async_copy(sbs[k], out_hbm.at[0, :, 0], sss[k]).wait()

    def transpose_add(g, k):
        gbuf = gbs[k]
        sbuf = sbs[k]
        ibuf = ibs[k]
        lsplat = jnp.full((16,), g, dtype=jnp.int32)
        hbv = [(ibuf[pl.ds(16 * j, 16)] & 1) * HID for j in range(8)]

        @plsc.parallel_loop(0, HID, step=1, unroll=4)
        def h_body(h):
            colsplat = jnp.full((16,), h, dtype=jnp.int32)
            pv = plsc.load_gather(pos_v, [lsplat, colsplat])
            th = h >> 3
            off = (h & 7) * 128
            for j in range(8):
                v = plsc.load_gather(gbuf, [rows[j], hbv[j] + colsplat])
                sbuf[th, pl.ds(off + 16 * j, 16)] = v + pv

    issue_idx(0, 0)
    issue_idx(1, 1)
    wait_idx(0)
    issue_gather(0)

    def outer(t, carry):
        for b in range(NBUF):
            g = t * NBUF + b
            k1 = (b + 1) % NBUF
            k2 = (b + 2) % NBUF

            def stage1():
                wait_idx(k1)
                issue_gather(k1)

            if b == 3:
                @pl.when(t < NT - 1)
                def _():
                    stage1()
            else:
                stage1()

            def stage2():
                issue_idx(g + 2, k2)

            if b >= 2:
                @pl.when(t < NT - 1)
                def _():
                    stage2()
            else:
                stage2()

            wait_gather(b)

            @pl.when(t > 0)
            def _():
                wait_store(b)

            transpose_add(g, b)
            issue_store(g, b)
        return carry

    lax.fori_loop(0, NT, outer, 0)

    for b in range(NBUF):
        wait_store(b)


def kernel(x, token_table, pos_table):
    xT = jnp.transpose(x).astype(jnp.int32)
    tok2 = token_table.reshape(VOCAB_PAIRS, 2 * HID)
    o = _emb_kernel(xT, tok2, pos_table)
    o = o.reshape(L, HID // 8, NW, 8, 128)
    return o.transpose(2, 4, 0, 1, 3).reshape(B, L, HID)


# speedup vs baseline: 1.0345x; 1.0345x over previous
import functools

import jax
import jax.numpy as jnp
from jax import lax
from jax.experimental import pallas as pl
from jax.experimental.pallas import tpu as pltpu
from jax.experimental.pallas import tpu_sc as plsc

B = 4096
L = 200
HID = 64

NC = 2
NS = 16
NW = NC * NS
BB = B // NW
NBUF = 4
NT = L // NBUF


@functools.partial(
    pl.kernel,
    mesh=plsc.VectorSubcoreMesh(core_axis_name="c", subcore_axis_name="s"),
    compiler_params=pltpu.CompilerParams(
        use_tc_tiling_on_sc=False, needs_layout_passes=False),
    out_type=jax.ShapeDtypeStruct((L, HID // 8, NW, 8 * 128), jnp.float32),
    scratch_types=[
        pltpu.VMEM((L, HID), jnp.float32),
        pltpu.VMEM((BB,), jnp.int32),
        pltpu.VMEM((BB,), jnp.int32),
        pltpu.VMEM((BB,), jnp.int32),
        pltpu.VMEM((BB,), jnp.int32),
        pltpu.VMEM((BB, HID), jnp.float32),
        pltpu.VMEM((BB, HID), jnp.float32),
        pltpu.VMEM((BB, HID), jnp.float32),
        pltpu.VMEM((BB, HID), jnp.float32),
        pltpu.VMEM((HID // 8, 8 * 128), jnp.float32),
        pltpu.VMEM((HID // 8, 8 * 128), jnp.float32),
        pltpu.VMEM((HID // 8, 8 * 128), jnp.float32),
        pltpu.VMEM((HID // 8, 8 * 128), jnp.float32),
        pltpu.SemaphoreType.DMA,
        pltpu.SemaphoreType.DMA,
        pltpu.SemaphoreType.DMA,
        pltpu.SemaphoreType.DMA,
        pltpu.SemaphoreType.DMA,
        pltpu.SemaphoreType.DMA,
        pltpu.SemaphoreType.DMA,
        pltpu.SemaphoreType.DMA,
        pltpu.SemaphoreType.DMA,
        pltpu.SemaphoreType.DMA,
        pltpu.SemaphoreType.DMA,
        pltpu.SemaphoreType.DMA,
    ],
)
def _emb_kernel(xT_hbm, tok_hbm, pos_hbm, out_hbm,
                pos_v,
                ib0, ib1, ib2, ib3,
                gb0, gb1, gb2, gb3,
                sb0, sb1, sb2, sb3,
                si0, si1, si2, si3,
                sg0, sg1, sg2, sg3,
                ss0, ss1, ss2, ss3):
    ibs = (ib0, ib1, ib2, ib3)
    gbs = (gb0, gb1, gb2, gb3)
    sbs = (sb0, sb1, sb2, sb3)
    sis = (si0, si1, si2, si3)
    sgs = (sg0, sg1, sg2, sg3)
    sss = (ss0, ss1, ss2, ss3)

    wid = lax.axis_index("s") * NC + lax.axis_index("c")
    col0 = wid * BB

    pltpu.sync_copy(pos_hbm.at[pl.ds(0, L)], pos_v)

    iota = lax.iota(jnp.int32, 16)
    rows = [iota + 16 * j for j in range(8)]

    def issue_idx(g, k):
        pltpu.async_copy(xT_hbm.at[g, pl.ds(col0, BB)], ibs[k], sis[k])

    def wait_idx(k):
        pltpu.make_async_copy(
            xT_hbm.at[0, pl.ds(0, BB)], ibs[k], sis[k]).wait()

    def issue_gather(k):
        pltpu.async_copy(tok_hbm.at[ibs[k]], gbs[k], sgs[k])

    def wait_gather(k):
        pltpu.make_async_copy(
            tok_hbm.at[pl.ds(0, BB)], gbs[k], sgs[k]).wait()

    def issue_store(g, k):
        pltpu.async_copy(sbs[k], out_hbm.at[g, :, wid], sss[k])

    def wait_store(k):
        pltpu.make_async_copy(sbs[k], out_hbm.at[0, :, 0], sss[k]).wait()

    def transpose_add(g, k):
        gbuf = gbs[k]
        sbuf = sbs[k]
        lsplat = jnp.full((16,), g, dtype=jnp.int32)

        @plsc.parallel_loop(0, HID, step=1, unroll=4)
        def h_body(h):
            colsplat = jnp.full((16,), h, dtype=jnp.int32)
            pv = plsc.load_gather(pos_v, [lsplat, colsplat])
            th = h >> 3
            off = (h & 7) * 128
            for j in range(8):
                v = plsc.load_gather(gbuf, [rows[j], colsplat])
                sbuf[th, pl.ds(off + 16 * j, 16)] = v + pv

    issue_idx(0, 0)
    issue_idx(1, 1)
    issue_idx(2, 2)
    wait_idx(0)
    issue_gather(0)
    wait_idx(1)
    issue_gather(1)

    def outer(t, carry):
        for b in range(NBUF):
            g = t * NBUF + b
            k2 = (b + 2) % NBUF
            k3 = (b + 3) % NBUF

            def prefetch_gather():
                wait_idx(k2)
                issue_gather(k2)

            if b >= 2:
                @pl.when(t < NT - 1)
                def _():
                    prefetch_gather()
            else:
                prefetch_gather()

            def prefetch_idx():
                issue_idx(g + 3, k3)

            if b == 0:
                prefetch_idx()
            else:
                @pl.when(t < NT - 1)
                def _():
                    prefetch_idx()

            wait_gather(b)

            @pl.when(t > 0)
            def _():
                wait_store(b)

            transpose_add(g, b)
            issue_store(g, b)
        return carry

    lax.fori_loop(0, NT, outer, 0)

    for b in range(NBUF):
        wait_store(b)


def kernel(x, token_table, pos_table):
    xT = jnp.transpose(x).astype(jnp.int32)
    o = _emb_kernel(xT, token_table, pos_table)
    o = o.reshape(L, HID // 8, NW, 8, 128)
    return o.transpose(2, 4, 0, 1, 3).reshape(B, L, HID)


# speedup vs baseline: 1.0352x; 1.0007x over previous
import functools

import jax
import jax.numpy as jnp
from jax import lax
from jax.experimental import pallas as pl
from jax.experimental.pallas import tpu as pltpu
from jax.experimental.pallas import tpu_sc as plsc

B = 4096
L = 200
HID = 64

NC = 2
NS = 16
NW = NC * NS
BB = B // NW
NBUF = 4
NT = L // NBUF


@functools.partial(
    pl.kernel,
    mesh=plsc.VectorSubcoreMesh(core_axis_name="c", subcore_axis_name="s"),
    compiler_params=pltpu.CompilerParams(
        use_tc_tiling_on_sc=False, needs_layout_passes=False),
    out_type=jax.ShapeDtypeStruct((L, HID // 8, NW, 8 * 128), jnp.float32),
    scratch_types=[
        pltpu.VMEM((L, HID), jnp.float32),
        pltpu.VMEM((L, BB), jnp.int32),
        pltpu.VMEM((BB, HID), jnp.float32),
        pltpu.VMEM((BB, HID), jnp.float32),
        pltpu.VMEM((BB, HID), jnp.float32),
        pltpu.VMEM((BB, HID), jnp.float32),
        pltpu.VMEM((HID // 8, 8 * 128), jnp.float32),
        pltpu.VMEM((HID // 8, 8 * 128), jnp.float32),
        pltpu.VMEM((HID // 8, 8 * 128), jnp.float32),
        pltpu.VMEM((HID // 8, 8 * 128), jnp.float32),
        pltpu.SemaphoreType.DMA,
        pltpu.SemaphoreType.DMA,
        pltpu.SemaphoreType.DMA,
        pltpu.SemaphoreType.DMA,
        pltpu.SemaphoreType.DMA,
        pltpu.SemaphoreType.DMA,
        pltpu.SemaphoreType.DMA,
        pltpu.SemaphoreType.DMA,
    ],
)
def _emb_kernel(xT_hbm, tok_hbm, pos_hbm, out_hbm,
                pos_v, idx_v,
                gb0, gb1, gb2, gb3,
                sb0, sb1, sb2, sb3,
                sg0, sg1, sg2, sg3,
                ss0, ss1, ss2, ss3):
    gbs = (gb0, gb1, gb2, gb3)
    sbs = (sb0, sb1, sb2, sb3)
    sgs = (sg0, sg1, sg2, sg3)
    sss = (ss0, ss1, ss2, ss3)

    wid = lax.axis_index("s") * NC + lax.axis_index("c")
    col0 = wid * BB

    pltpu.sync_copy(pos_hbm.at[pl.ds(0, L)], pos_v)
    pltpu.sync_copy(xT_hbm.at[:, pl.ds(col0, BB)], idx_v)

    iota = lax.iota(jnp.int32, 16)
    rows = [iota + 16 * j for j in range(8)]

    def issue_gather(g, k):
        pltpu.async_copy(tok_hbm.at[idx_v.at[g]], gbs[k], sgs[k])

    def wait_gather(k):
        pltpu.make_async_copy(
            tok_hbm.at[pl.ds(0, BB)], gbs[k], sgs[k]).wait()

    def issue_store(g, k):
        pltpu.async_copy(sbs[k], out_hbm.at[g, :, wid], sss[k])

    def wait_store(k):
        pltpu.make_async_copy(sbs[k], out_hbm.at[0, :, 0], sss[k]).wait()

    def transpose_add(g, k):
        gbuf = gbs[k]
        sbuf = sbs[k]
        lsplat = jnp.full((16,), g, dtype=jnp.int32)

        @plsc.parallel_loop(0, HID, step=1, unroll=4)
        def h_body(h):
            colsplat = jnp.full((16,), h, dtype=jnp.int32)
            pv = plsc.load_gather(pos_v, [lsplat, colsplat])
            th = h >> 3
            off = (h & 7) * 128
            for j in range(8):
                v = plsc.load_gather(gbuf, [rows[j], colsplat])
                sbuf[th, pl.ds(off + 16 * j, 16)] = v + pv

    issue_gather(0, 0)
    issue_gather(1, 1)
    issue_gather(2, 2)

    def outer(t, carry):
        for b in range(NBUF):
            g = t * NBUF + b
            k3 = (b + 3) % NBUF

            def prefetch_gather():
                issue_gather(g + 3, k3)

            if b == 0:
                prefetch_gather()
            else:
                @pl.when(t < NT - 1)
                def _():
                    prefetch_gather()

            wait_gather(b)

            @pl.when(t > 0)
            def _():
                wait_store(b)

            transpose_add(g, b)
            issue_store(g, b)
        return carry

    lax.fori_loop(0, NT, outer, 0)

    for b in range(NBUF):
        wait_store(b)


def kernel(x, token_table, pos_table):
    xT = jnp.transpose(x).astype(jnp.int32)
    o = _emb_kernel(xT, token_table, pos_table)
    o = o.reshape(L, HID // 8, NW, 8, 128)
    return o.transpose(2, 4, 0, 1, 3).reshape(B, L, HID)
